# baseline (speedup 1.0000x reference)
import functools
import math

import jax
import jax.numpy as jnp
from jax import lax
from jax.experimental import pallas as pl
from jax.experimental.pallas import tpu as pltpu
from jax.experimental.pallas import tpu_sc as plsc

D_MODEL = 1024
SCALE = math.sqrt(D_MODEL)
LANES = 16
NW = 32
CH = 32
NBUF = 3
LOOKAHEAD = 1


def _embed(idx, table):
    (B,) = idx.shape
    V, D = table.shape
    b_per_w = B // NW
    n_ch = b_per_w // CH
    n_grp = n_ch // NBUF

    mesh = plsc.VectorSubcoreMesh(core_axis_name="c", subcore_axis_name="s")

    @functools.partial(
        pl.kernel,
        out_type=jax.ShapeDtypeStruct((B, D), jnp.float32),
        mesh=mesh,
        scratch_types=[
            pltpu.VMEM((b_per_w,), jnp.int32),
            pltpu.VMEM((NBUF, CH, D), jnp.float32),
            pltpu.SemaphoreType.DMA((NBUF,)),
            pltpu.SemaphoreType.DMA((NBUF,)),
        ],
    )
    def emb(table_hbm, idx_hbm, out_hbm, idx_v, bufs, gsems, osems):
        wid = lax.axis_index("s") * 2 + lax.axis_index("c")
        base = wid * b_per_w
        pltpu.sync_copy(idx_hbm.at[pl.ds(base, b_per_w)], idx_v)

        def gather_start(c, j):
            pltpu.async_copy(
                table_hbm.at[idx_v.at[pl.ds(c * CH, CH)]],
                bufs.at[j],
                gsems.at[j],
            )

        def gather_wait(j):
            pltpu.make_async_copy(
                table_hbm.at[pl.ds(0, CH)], bufs.at[j], gsems.at[j]
            ).wait()

        def wb_start(c, j):
            pltpu.async_copy(
                bufs.at[j], out_hbm.at[pl.ds(base + c * CH, CH)], osems.at[j]
            )

        def wb_wait(j):
            pltpu.make_async_copy(
                bufs.at[j], out_hbm.at[pl.ds(0, CH)], osems.at[j]
            ).wait()

        def scale_buf(j):
            b = bufs.at[j]

            @plsc.parallel_loop(0, CH)
            def _(r):
                for k in range(D // LANES):
                    sl = pl.ds(k * LANES, LANES)
                    b[r, sl] = b[r, sl] * SCALE

        for c0 in range(LOOKAHEAD):
            gather_start(c0, c0 % NBUF)
        for c in range(n_ch):
            cg = c + LOOKAHEAD
            if cg < n_ch:
                jg = cg % NBUF
                if cg >= NBUF:
                    wb_wait(jg)
                gather_start(cg, jg)
            gather_wait(c % NBUF)
            scale_buf(c % NBUF)
            wb_start(c, c % NBUF)
        for jj in range(NBUF):
            wb_wait(jj)

    return emb(table, idx)


def kernel(x, table):
    B0, S = x.shape
    idx = x.reshape(B0 * S)
    out = _embed(idx, table)
    return out.reshape(B0, S, table.shape[1])


# speedup vs baseline: 1.0094x; 1.0094x over previous
import functools
import math

import jax
import jax.numpy as jnp
from jax import lax
from jax.experimental import pallas as pl
from jax.experimental.pallas import tpu as pltpu
from jax.experimental.pallas import tpu_sc as plsc

D_MODEL = 1024
SCALE = math.sqrt(D_MODEL)
LANES = 16
NW = 32
CH = 8
NBUF = 8
LOOKAHEAD = 4


def _embed(idx, table):
    (B,) = idx.shape
    V, D = table.shape
    b_per_w = B // NW
    n_ch = b_per_w // CH
    n_grp = n_ch // NBUF

    mesh = plsc.VectorSubcoreMesh(core_axis_name="c", subcore_axis_name="s")

    @functools.partial(
        pl.kernel,
        out_type=jax.ShapeDtypeStruct((B, D), jnp.float32),
        mesh=mesh,
        scratch_types=[
            pltpu.VMEM((b_per_w,), jnp.int32),
            pltpu.VMEM((NBUF, CH, D), jnp.float32),
            pltpu.SemaphoreType.DMA((NBUF,)),
            pltpu.SemaphoreType.DMA((NBUF,)),
        ],
    )
    def emb(table_hbm, idx_hbm, out_hbm, idx_v, bufs, gsems, osems):
        wid = lax.axis_index("s") * 2 + lax.axis_index("c")
        base = wid * b_per_w
        pltpu.sync_copy(idx_hbm.at[pl.ds(base, b_per_w)], idx_v)

        def gather_start(c, j):
            pltpu.async_copy(
                table_hbm.at[idx_v.at[pl.ds(c * CH, CH)]],
                bufs.at[j],
                gsems.at[j],
            )

        def gather_wait(j):
            pltpu.make_async_copy(
                table_hbm.at[pl.ds(0, CH)], bufs.at[j], gsems.at[j]
            ).wait()

        def wb_start(c, j):
            pltpu.async_copy(
                bufs.at[j], out_hbm.at[pl.ds(base + c * CH, CH)], osems.at[j]
            )

        def wb_wait(j):
            pltpu.make_async_copy(
                bufs.at[j], out_hbm.at[pl.ds(0, CH)], osems.at[j]
            ).wait()

        def scale_buf(j):
            b = bufs.at[j]

            @plsc.parallel_loop(0, CH)
            def _(r):
                for k in range(D // LANES):
                    sl = pl.ds(k * LANES, LANES)
                    b[r, sl] = b[r, sl] * SCALE

        def step(c, jj, wb_pending, do_gather=True):
            jg = (jj + LOOKAHEAD) % NBUF
            if do_gather:
                if wb_pending:
                    wb_wait(jg)
                gather_start(c + LOOKAHEAD, jg)
            gather_wait(jj)
            scale_buf(jj)
            wb_start(c, jj)

        for c0 in range(LOOKAHEAD):
            gather_start(c0, c0 % NBUF)

        for jj in range(NBUF):
            step(jj, jj, wb_pending=jj + LOOKAHEAD >= NBUF)

        def body(g, _):
            c_base = g * NBUF
            for jj in range(NBUF):
                step(c_base + jj, jj, wb_pending=True)
            return 0

        lax.fori_loop(1, n_grp - 1, body, 0)

        c_base = (n_grp - 1) * NBUF
        for jj in range(NBUF):
            step(
                c_base + jj,
                jj,
                wb_pending=True,
                do_gather=jj + LOOKAHEAD < NBUF,
            )

        for jj in range(NBUF):
            wb_wait(jj)

    return emb(table, idx)


def kernel(x, table):
    B0, S = x.shape
    idx = x.reshape(B0 * S)
    out = _embed(idx, table)
    return out.reshape(B0, S, table.shape[1])


# speedup vs baseline: 1.0476x; 1.0379x over previous
import functools
import math

import jax
import jax.numpy as jnp
from jax import lax
from jax.experimental import pallas as pl
from jax.experimental.pallas import tpu as pltpu
from jax.experimental.pallas import tpu_sc as plsc

D_MODEL = 1024
SCALE = math.sqrt(D_MODEL)
LANES = 16
NW = 32
CH = 16
NBUF = 4
LOOKAHEAD = 2


def kernel(x, table):
    B0, S = x.shape
    V, D = table.shape
    x = x.astype(jnp.int32)
    b_per_w = (B0 * S) // NW
    w_per_row = S // b_per_w
    n_ch = b_per_w // CH
    n_grp = n_ch // NBUF

    mesh = plsc.VectorSubcoreMesh(core_axis_name="c", subcore_axis_name="s")

    @functools.partial(
        pl.kernel,
        out_type=jax.ShapeDtypeStruct((B0, S, D), jnp.float32),
        mesh=mesh,
        scratch_types=[
            pltpu.VMEM((b_per_w,), jnp.int32),
            pltpu.VMEM((NBUF, CH, D), jnp.float32),
            pltpu.SemaphoreType.DMA((NBUF,)),
            pltpu.SemaphoreType.DMA((NBUF,)),
        ],
    )
    def emb(table_hbm, idx_hbm, out_hbm, idx_v, bufs, gsems, osems):
        wid = lax.axis_index("s") * 2 + lax.axis_index("c")
        row = wid // w_per_row
        col = (wid % w_per_row) * b_per_w
        pltpu.sync_copy(idx_hbm.at[row, pl.ds(col, b_per_w)], idx_v)

        def gather_start(c, j):
            pltpu.async_copy(
                table_hbm.at[idx_v.at[pl.ds(c * CH, CH)]],
                bufs.at[j],
                gsems.at[j],
            )

        def gather_wait(j):
            pltpu.make_async_copy(
                table_hbm.at[pl.ds(0, CH)], bufs.at[j], gsems.at[j]
            ).wait()

        def wb_start(c, j):
            pltpu.async_copy(
                bufs.at[j],
                out_hbm.at[row, pl.ds(col + c * CH, CH)],
                osems.at[j],
            )

        def wb_wait(j):
            pltpu.make_async_copy(
                bufs.at[j], out_hbm.at[0, pl.ds(0, CH)], osems.at[j]
            ).wait()

        def scale_buf(j):
            b = bufs.at[j]

            @plsc.parallel_loop(0, CH)
            def _(r):
                for k in range(D // LANES):
                    sl = pl.ds(k * LANES, LANES)
                    b[r, sl] = b[r, sl] * SCALE

        def step(c, jj, wb_pending, do_gather=True):
            jg = (jj + LOOKAHEAD) % NBUF
            if do_gather:
                if wb_pending:
                    wb_wait(jg)
                gather_start(c + LOOKAHEAD, jg)
            gather_wait(jj)
            scale_buf(jj)
            wb_start(c, jj)

        for c0 in range(LOOKAHEAD):
            gather_start(c0, c0 % NBUF)

        for jj in range(NBUF):
            step(jj, jj, wb_pending=jj + LOOKAHEAD >= NBUF)

        def body(g, _):
            c_base = g * NBUF
            for jj in range(NBUF):
                step(c_base + jj, jj, wb_pending=True)
            return 0

        lax.fori_loop(1, n_grp - 1, body, 0)

        c_base = (n_grp - 1) * NBUF
        for jj in range(NBUF):
            step(
                c_base + jj,
                jj,
                wb_pending=True,
                do_gather=jj + LOOKAHEAD < NBUF,
            )

        for jj in range(NBUF):
            wb_wait(jj)

    return emb(table, x)


# speedup vs baseline: 1.1457x; 1.0936x over previous
import functools
import math

import jax
import jax.numpy as jnp
from jax import lax
from jax.experimental import pallas as pl
from jax.experimental.pallas import tpu as pltpu
from jax.experimental.pallas import tpu_sc as plsc

D_MODEL = 1024
SCALE = math.sqrt(D_MODEL)
LANES = 16
NW = 32
CH = 16
NBUF = 4
LOOKAHEAD = 2


def kernel(x, table):
    B0, S = x.shape
    V, D = table.shape
    x = x.astype(jnp.int32)
    b_per_w = (B0 * S) // NW
    w_per_row = S // b_per_w
    n_ch = b_per_w // CH

    mesh = plsc.VectorSubcoreMesh(core_axis_name="c", subcore_axis_name="s")

    @functools.partial(
        pl.kernel,
        out_type=jax.ShapeDtypeStruct((B0, S, D), jnp.float32),
        mesh=mesh,
        scratch_types=[
            pltpu.VMEM((b_per_w,), jnp.int32),
            pltpu.VMEM((NBUF * CH, D), jnp.float32),
            pltpu.SemaphoreType.DMA((NBUF,)),
            pltpu.SemaphoreType.DMA((NBUF,)),
        ],
    )
    def emb(table_hbm, idx_hbm, out_hbm, idx_v, bufs, gsems, osems):
        wid = lax.axis_index("s") * 2 + lax.axis_index("c")
        row = wid // w_per_row
        col = (wid % w_per_row) * b_per_w
        pltpu.sync_copy(idx_hbm.at[row, pl.ds(col, b_per_w)], idx_v)

        def gather_start(c, j):
            pltpu.async_copy(
                table_hbm.at[idx_v.at[pl.ds(c * CH, CH)]],
                bufs.at[pl.ds(j * CH, CH)],
                gsems.at[j],
            )

        def gather_wait(j):
            pltpu.make_async_copy(
                table_hbm.at[pl.ds(0, CH)],
                bufs.at[pl.ds(j * CH, CH)],
                gsems.at[j],
            ).wait()

        def wb_start(c, j):
            pltpu.async_copy(
                bufs.at[pl.ds(j * CH, CH)],
                out_hbm.at[row, pl.ds(col + c * CH, CH)],
                osems.at[j],
            )

        def wb_wait(j):
            pltpu.make_async_copy(
                bufs.at[pl.ds(j * CH, CH)],
                out_hbm.at[0, pl.ds(0, CH)],
                osems.at[j],
            ).wait()

        for c0 in range(LOOKAHEAD):
            gather_start(c0, c0)

        def body(c, _):
            j = lax.rem(c, NBUF)
            cg = c + LOOKAHEAD

            @pl.when(cg < n_ch)
            def _():
                jg = lax.rem(cg, NBUF)

                @pl.when(cg >= NBUF)
                def _():
                    wb_wait(jg)

                gather_start(cg, jg)

            gather_wait(j)

            rbase = j * CH

            @plsc.parallel_loop(0, CH)
            def _(r):
                for k in range(D // LANES):
                    sl = pl.ds(k * LANES, LANES)
                    bufs[rbase + r, sl] = bufs[rbase + r, sl] * SCALE

            wb_start(c, j)
            return 0

        lax.fori_loop(0, n_ch, body, 0)

        for jj in range(NBUF):
            wb_wait(jj)

    return emb(table, x)


# speedup vs baseline: 1.1554x; 1.0085x over previous
import functools
import math

import jax
import jax.numpy as jnp
from jax import lax
from jax.experimental import pallas as pl
from jax.experimental.pallas import tpu as pltpu
from jax.experimental.pallas import tpu_sc as plsc

D_MODEL = 1024
SCALE = math.sqrt(D_MODEL)
LANES = 16
NW = 32
CH = 16
NBUF = 5
LOOKAHEAD = 3


def kernel(x, table):
    B0, S = x.shape
    V, D = table.shape
    x = x.astype(jnp.int32)
    b_per_w = (B0 * S) // NW
    w_per_row = S // b_per_w
    n_ch = b_per_w // CH

    mesh = plsc.VectorSubcoreMesh(core_axis_name="c", subcore_axis_name="s")

    @functools.partial(
        pl.kernel,
        out_type=jax.ShapeDtypeStruct((B0, S, D), jnp.float32),
        mesh=mesh,
        scratch_types=[
            pltpu.VMEM((b_per_w,), jnp.int32),
            pltpu.VMEM((NBUF * CH, D), jnp.float32),
            pltpu.SemaphoreType.DMA((NBUF,)),
            pltpu.SemaphoreType.DMA((NBUF,)),
        ],
    )
    def emb(table_hbm, idx_hbm, out_hbm, idx_v, bufs, gsems, osems):
        wid = lax.axis_index("s") * 2 + lax.axis_index("c")
        row = wid // w_per_row
        col = (wid % w_per_row) * b_per_w
        pltpu.sync_copy(idx_hbm.at[row, pl.ds(col, b_per_w)], idx_v)

        def gather_start(c, j):
            pltpu.async_copy(
                table_hbm.at[idx_v.at[pl.ds(c * CH, CH)]],
                bufs.at[pl.ds(j * CH, CH)],
                gsems.at[j],
            )

        def gather_wait(j):
            pltpu.make_async_copy(
                table_hbm.at[pl.ds(0, CH)],
                bufs.at[pl.ds(j * CH, CH)],
                gsems.at[j],
            ).wait()

        def wb_start(c, j):
            pltpu.async_copy(
                bufs.at[pl.ds(j * CH, CH)],
                out_hbm.at[row, pl.ds(col + c * CH, CH)],
                osems.at[j],
            )

        def wb_wait(j):
            pltpu.make_async_copy(
                bufs.at[pl.ds(j * CH, CH)],
                out_hbm.at[0, pl.ds(0, CH)],
                osems.at[j],
            ).wait()

        for c0 in range(LOOKAHEAD):
            gather_start(c0, c0)

        def body(c, _):
            j = lax.rem(c, NBUF)
            cg = c + LOOKAHEAD

            @pl.when(cg < n_ch)
            def _():
                jg = lax.rem(cg, NBUF)

                @pl.when(cg >= NBUF)
                def _():
                    wb_wait(jg)

                gather_start(cg, jg)

            gather_wait(j)

            rbase = j * CH

            @plsc.parallel_loop(0, CH)
            def _(r):
                for k in range(D // LANES):
                    sl = pl.ds(k * LANES, LANES)
                    bufs[rbase + r, sl] = bufs[rbase + r, sl] * SCALE

            wb_start(c, j)
            return 0

        lax.fori_loop(0, n_ch, body, 0)

        for jj in range(NBUF):
            wb_wait(jj)

    return emb(table, x)


# speedup vs baseline: 1.1603x; 1.0042x over previous
import functools
import math

import jax
import jax.numpy as jnp
from jax import lax
from jax.experimental import pallas as pl
from jax.experimental.pallas import tpu as pltpu
from jax.experimental.pallas import tpu_sc as plsc

D_MODEL = 1024
SCALE = math.sqrt(D_MODEL)
LANES = 16
NW = 32
CH = 16
NBUF = 6
LOOKAHEAD = 4


def kernel(x, table):
    B0, S = x.shape
    V, D = table.shape
    x = x.astype(jnp.int32)
    b_per_w = (B0 * S) // NW
    w_per_row = S // b_per_w
    n_ch = b_per_w // CH

    mesh = plsc.VectorSubcoreMesh(core_axis_name="c", subcore_axis_name="s")

    @functools.partial(
        pl.kernel,
        out_type=jax.ShapeDtypeStruct((B0, S, D), jnp.float32),
        mesh=mesh,
        scratch_types=[
            pltpu.VMEM((b_per_w,), jnp.int32),
            pltpu.VMEM((NBUF * CH, D), jnp.float32),
            pltpu.SemaphoreType.DMA((NBUF,)),
            pltpu.SemaphoreType.DMA((NBUF,)),
        ],
    )
    def emb(table_hbm, idx_hbm, out_hbm, idx_v, bufs, gsems, osems):
        wid = lax.axis_index("s") * 2 + lax.axis_index("c")
        row = wid // w_per_row
        col = (wid % w_per_row) * b_per_w
        pltpu.sync_copy(idx_hbm.at[row, pl.ds(col, b_per_w)], idx_v)

        def gather_start(c, j):
            pltpu.async_copy(
                table_hbm.at[idx_v.at[pl.ds(c * CH, CH)]],
                bufs.at[pl.ds(j * CH, CH)],
                gsems.at[j],
            )

        def gather_wait(j):
            pltpu.make_async_copy(
                table_hbm.at[pl.ds(0, CH)],
                bufs.at[pl.ds(j * CH, CH)],
                gsems.at[j],
            ).wait()

        def wb_start(c, j):
            pltpu.async_copy(
                bufs.at[pl.ds(j * CH, CH)],
                out_hbm.at[row, pl.ds(col + c * CH, CH)],
                osems.at[j],
            )

        def wb_wait(j):
            pltpu.make_async_copy(
                bufs.at[pl.ds(j * CH, CH)],
                out_hbm.at[0, pl.ds(0, CH)],
                osems.at[j],
            ).wait()

        for c0 in range(LOOKAHEAD):
            gather_start(c0, c0)

        def body(c, _):
            j = lax.rem(c, NBUF)
            cg = c + LOOKAHEAD

            @pl.when(cg < n_ch)
            def _():
                jg = lax.rem(cg, NBUF)

                @pl.when(cg >= NBUF)
                def _():
                    wb_wait(jg)

                gather_start(cg, jg)

            gather_wait(j)

            rbase = j * CH

            @plsc.parallel_loop(0, CH)
            def _(r):
                for k in range(D // LANES):
                    sl = pl.ds(k * LANES, LANES)
                    bufs[rbase + r, sl] = bufs[rbase + r, sl] * SCALE

            wb_start(c, j)
            return 0

        lax.fori_loop(0, n_ch, body, 0)

        for jj in range(NBUF):
            wb_wait(jj)

    return emb(table, x)
